# baseline (speedup 1.0000x reference)
import functools

import jax
import jax.numpy as jnp
from jax import lax
from jax.experimental import pallas as pl
from jax.experimental.pallas import tpu as pltpu, tpu_sc as plsc

NUM_WORKERS = 32
BATCH = 16384
DIM = 64
ROWS_PER_WORKER = BATCH // NUM_WORKERS
CHUNK = 128
NCHUNK = ROWS_PER_WORKER // CHUNK
GROUPS_PER_CHUNK = CHUNK // 16


_GATHER_DNUMS = lax.GatherDimensionNumbers(
    offset_dims=(), collapsed_slice_dims=(0,), start_index_map=(0,))


def _lane_permute(v, idx):
    return lax.gather(v, idx[:, None], _GATHER_DNUMS, (1,),
                      mode=lax.GatherScatterMode.PROMISE_IN_BOUNDS)


def _hsum_butterfly(v, iota16):
    for sh in (8, 4, 2, 1):
        v = v + _lane_permute(v, iota16 ^ sh)
    return v


def _rsqrt_newton(x):
    i = lax.bitcast_convert_type(x, jnp.int32)
    i = jnp.int32(0x5F3759DF) - lax.shift_right_arithmetic(i, 1)
    y = lax.bitcast_convert_type(i, jnp.float32)
    for _ in range(3):
        y = y * (1.5 - 0.5 * x * y * y)
    return y


def _body(ent_hbm, rel_hbm, h_hbm, r_hbm, t_hbm, out_hbm,
          hidx, ridx, tidx, headb, relb, tailb, outb,
          sem0, sem1, sem2, sem3):
    sems = (sem0, sem1, sem2, sem3)
    wid = lax.axis_index("s") * 2 + lax.axis_index("c")
    base = wid * ROWS_PER_WORKER

    pltpu.sync_copy(h_hbm.at[pl.ds(wid * NCHUNK, NCHUNK)], hidx)
    pltpu.sync_copy(r_hbm.at[pl.ds(wid * NCHUNK, NCHUNK)], ridx)
    pltpu.sync_copy(t_hbm.at[pl.ds(wid * NCHUNK, NCHUNK)], tidx)

    copies = []
    for c in range(NCHUNK):
        dst = pl.ds(c * CHUNK, CHUNK)
        copies.append((
            pltpu.async_copy(ent_hbm.at[hidx.at[c]], headb.at[dst], sems[c]),
            pltpu.async_copy(rel_hbm.at[ridx.at[c]], relb.at[dst], sems[c]),
            pltpu.async_copy(ent_hbm.at[tidx.at[c]], tailb.at[dst], sems[c]),
        ))

    iota16 = lax.iota(jnp.int32, 16)

    def group_step(g, carry):
        s_vec = jnp.zeros((16,), jnp.float32)
        for k in range(16):
            i = g * 16 + k
            acc = jnp.zeros((16,), jnp.float32)
            for q in range(DIM // 16):
                hv = headb[i, pl.ds(q * 16, 16)]
                rv = relb[i, pl.ds(q * 16, 16)]
                tv = tailb[i, pl.ds(q * 16, 16)]
                d = hv + rv - tv
                acc = acc + d * d
            s_vec = jnp.where(iota16 == k, _hsum_butterfly(acc, iota16), s_vec)
        x = s_vec + 1e-12
        outb[pl.ds(g * 16, 16)] = x * _rsqrt_newton(x)
        return carry

    for c in range(NCHUNK):
        for cp in copies[c]:
            cp.wait()
        lax.fori_loop(c * (CHUNK // 16), (c + 1) * (CHUNK // 16), group_step, 0)

    pltpu.sync_copy(outb, out_hbm.at[pl.ds(base, ROWS_PER_WORKER)])


@jax.jit
def _sc_transe(emb_ent, emb_rel, h2, r2, t2):
    mesh = plsc.VectorSubcoreMesh(core_axis_name="c", subcore_axis_name="s",
                                  num_cores=2, num_subcores=16)
    f = pl.kernel(
        _body,
        out_type=jax.ShapeDtypeStruct((BATCH,), jnp.float32),
        mesh=mesh,
        compiler_params=pltpu.CompilerParams(
            use_tc_tiling_on_sc=False,
            needs_layout_passes=False,
        ),
        scratch_types=[
            pltpu.VMEM((NCHUNK, CHUNK), jnp.int32),
            pltpu.VMEM((NCHUNK, CHUNK), jnp.int32),
            pltpu.VMEM((NCHUNK, CHUNK), jnp.int32),
            pltpu.VMEM((ROWS_PER_WORKER, DIM), jnp.float32),
            pltpu.VMEM((ROWS_PER_WORKER, DIM), jnp.float32),
            pltpu.VMEM((ROWS_PER_WORKER, DIM), jnp.float32),
            pltpu.VMEM((ROWS_PER_WORKER,), jnp.float32),
            pltpu.SemaphoreType.DMA,
            pltpu.SemaphoreType.DMA,
            pltpu.SemaphoreType.DMA,
            pltpu.SemaphoreType.DMA,
        ],
    )
    return f(emb_ent, emb_rel, h2, r2, t2)


def kernel(emb_ent, emb_rel, h, r, t):
    h2 = h.astype(jnp.int32).reshape(NUM_WORKERS * NCHUNK, CHUNK)
    r2 = r.astype(jnp.int32).reshape(NUM_WORKERS * NCHUNK, CHUNK)
    t2 = t.astype(jnp.int32).reshape(NUM_WORKERS * NCHUNK, CHUNK)
    return _sc_transe(emb_ent, emb_rel, h2, r2, t2)


# speedup vs baseline: 1.6848x; 1.6848x over previous
import jax
import jax.numpy as jnp
from jax import lax
from jax.experimental import pallas as pl
from jax.experimental.pallas import tpu as pltpu, tpu_sc as plsc

NUM_WORKERS = 32
BATCH = 16384
DIM = 64
ROWS_PER_WORKER = BATCH // NUM_WORKERS
CHUNK = 128
NCHUNK = ROWS_PER_WORKER // CHUNK
GROUPS_PER_CHUNK = CHUNK // 16

_GATHER_DNUMS = lax.GatherDimensionNumbers(
    offset_dims=(), collapsed_slice_dims=(0,), start_index_map=(0,))


def _lane_permute(v, idx):
    return lax.gather(v, idx[:, None], _GATHER_DNUMS, (1,),
                      mode=lax.GatherScatterMode.PROMISE_IN_BOUNDS)


def _hsum_butterfly(v, iota16):
    for sh in (8, 4, 2, 1):
        v = v + _lane_permute(v, iota16 ^ sh)
    return v


def _rsqrt_newton(x):
    i = lax.bitcast_convert_type(x, jnp.int32)
    i = jnp.int32(0x5F3759DF) - lax.shift_right_arithmetic(i, 1)
    y = lax.bitcast_convert_type(i, jnp.float32)
    for _ in range(3):
        y = y * (1.5 - 0.5 * x * y * y)
    return y


def _body(ent_hbm, rel_hbm, h_hbm, r_hbm, t_hbm, out_hbm,
          hidx, ridx, tidx, headb, relb, tailb, outb, sem0, sem1):
    sems = (sem0, sem1)
    wid = lax.axis_index("s") * 2 + lax.axis_index("c")
    base = wid * ROWS_PER_WORKER

    pltpu.sync_copy(h_hbm.at[pl.ds(base, ROWS_PER_WORKER)], hidx)
    pltpu.sync_copy(r_hbm.at[pl.ds(base, ROWS_PER_WORKER)], ridx)
    pltpu.sync_copy(t_hbm.at[pl.ds(base, ROWS_PER_WORKER)], tidx)

    def issue_chunk(c):
        buf = c % 2
        sem = sems[buf]

        def issue_group(g, carry):
            src_row0 = c * CHUNK + g * 16
            hv = hidx[pl.ds(src_row0, 16)]
            rv = ridx[pl.ds(src_row0, 16)]
            tv = tidx[pl.ds(src_row0, 16)]
            for k in range(16):
                dst = pl.ds(g * 16 + k, 1)
                pltpu.async_copy(ent_hbm.at[pl.ds(hv[k], 1)],
                                 headb.at[buf].at[dst], sem)
                pltpu.async_copy(rel_hbm.at[pl.ds(rv[k], 1)],
                                 relb.at[buf].at[dst], sem)
                pltpu.async_copy(ent_hbm.at[pl.ds(tv[k], 1)],
                                 tailb.at[buf].at[dst], sem)
            return carry

        lax.fori_loop(0, GROUPS_PER_CHUNK, issue_group, 0)

    def drain_chunk(c):
        buf = c % 2
        sem = sems[buf]
        pltpu.make_async_copy(ent_hbm.at[pl.ds(0, CHUNK)], headb.at[buf], sem).wait()
        pltpu.make_async_copy(rel_hbm.at[pl.ds(0, CHUNK)], relb.at[buf], sem).wait()
        pltpu.make_async_copy(ent_hbm.at[pl.ds(0, CHUNK)], tailb.at[buf], sem).wait()

    iota16 = lax.iota(jnp.int32, 16)

    def compute_chunk(c):
        buf = c % 2

        def group_step(g, carry):
            s_vec = jnp.zeros((16,), jnp.float32)
            for k in range(16):
                i = g * 16 + k
                acc = jnp.zeros((16,), jnp.float32)
                for q in range(DIM // 16):
                    hv = headb[buf, i, pl.ds(q * 16, 16)]
                    rv = relb[buf, i, pl.ds(q * 16, 16)]
                    tv = tailb[buf, i, pl.ds(q * 16, 16)]
                    d = hv + rv - tv
                    acc = acc + d * d
                s_vec = jnp.where(iota16 == k, _hsum_butterfly(acc, iota16), s_vec)
            x = s_vec + 1e-12
            outb[pl.ds(c * CHUNK + g * 16, 16)] = x * _rsqrt_newton(x)
            return carry

        lax.fori_loop(0, GROUPS_PER_CHUNK, group_step, 0)

    issue_chunk(0)
    for c in range(NCHUNK):
        if c + 1 < NCHUNK:
            issue_chunk(c + 1)
        drain_chunk(c)
        compute_chunk(c)

    pltpu.sync_copy(outb, out_hbm.at[pl.ds(base, ROWS_PER_WORKER)])


@jax.jit
def _sc_transe(emb_ent, emb_rel, h, r, t):
    mesh = plsc.VectorSubcoreMesh(core_axis_name="c", subcore_axis_name="s",
                                  num_cores=2, num_subcores=16)
    f = pl.kernel(
        _body,
        out_type=jax.ShapeDtypeStruct((BATCH,), jnp.float32),
        mesh=mesh,
        compiler_params=pltpu.CompilerParams(
            use_tc_tiling_on_sc=True,
            needs_layout_passes=False,
        ),
        scratch_types=[
            pltpu.VMEM((ROWS_PER_WORKER,), jnp.int32),
            pltpu.VMEM((ROWS_PER_WORKER,), jnp.int32),
            pltpu.VMEM((ROWS_PER_WORKER,), jnp.int32),
            pltpu.VMEM((2, CHUNK, DIM), jnp.float32),
            pltpu.VMEM((2, CHUNK, DIM), jnp.float32),
            pltpu.VMEM((2, CHUNK, DIM), jnp.float32),
            pltpu.VMEM((ROWS_PER_WORKER,), jnp.float32),
            pltpu.SemaphoreType.DMA,
            pltpu.SemaphoreType.DMA,
        ],
    )
    return f(emb_ent, emb_rel, h, r, t)


def kernel(emb_ent, emb_rel, h, r, t):
    return _sc_transe(emb_ent, emb_rel,
                      h.astype(jnp.int32), r.astype(jnp.int32),
                      t.astype(jnp.int32))


# speedup vs baseline: 3.5899x; 2.1308x over previous
import jax
import jax.numpy as jnp
from jax import lax
from jax.experimental import pallas as pl
from jax.experimental.pallas import tpu as pltpu, tpu_sc as plsc

BATCH = 16384
DIM = 64
NSC = 2
NSUB = 16
J_PER_SC = DIM // NSC
B_PER_SUB = BATCH // NSUB
NENT = 1000000
RELW = 1024
ROWBUF = NENT + RELW
IDX_CHUNK = 128
NCH = B_PER_SUB // IDX_CHUNK
GW = 3 * IDX_CHUNK


def _sc_body(ent_hbm, rel_hbm, h_hbm, r_hbm, t_hbm, p0_hbm, p1_hbm,
             ent0, ent1, rel0, rel1, hidx, tidx, ridx, gbuf, acc,
             sem_r0, sem_r1, sem_g0, sem_g1):
    c = lax.axis_index("c")
    s = lax.axis_index("s")
    jbase = c * J_PER_SC
    b0 = s * B_PER_SUB

    ent_bufs = (ent0, ent1)
    rel_bufs = (rel0, rel1)
    row_sems = (sem_r0, sem_r1)
    g_sems = (sem_g0, sem_g1)

    def issue_row(jj, buf, rbuf, sem):
        @pl.when(s == jj % NSUB)
        def _():
            pltpu.async_copy(ent_hbm.at[jbase + jj], buf, sem)
            pltpu.async_copy(rel_hbm.at[jbase + jj], rbuf, sem)

    def wait_row(jj, buf, rbuf, sem):
        @pl.when(s == jj % NSUB)
        def _():
            pltpu.make_async_copy(ent_hbm.at[0], buf, sem).wait()
            pltpu.make_async_copy(rel_hbm.at[0], rbuf, sem).wait()

    pltpu.sync_copy(h_hbm.at[pl.ds(b0, B_PER_SUB)], hidx)
    pltpu.sync_copy(t_hbm.at[pl.ds(b0, B_PER_SUB)], tidx)
    pltpu.sync_copy(r_hbm.at[pl.ds(b0, B_PER_SUB)], ridx)
    for i in range(B_PER_SUB // 16):
        d = pl.ds(i * 16, 16)
        acc[d] = jnp.zeros((16,), jnp.float32)
    issue_row(0, ent0, rel0, sem_r0)
    issue_row(1, ent1, rel1, sem_r1)

    def issue_chunk(buf, rbuf, ch, slot):
        d = pl.ds(ch * IDX_CHUNK, IDX_CHUNK)
        base = slot * GW
        sem = g_sems[slot]
        pltpu.async_copy(buf.at[hidx.at[d]], gbuf.at[pl.ds(base, IDX_CHUNK)], sem)
        pltpu.async_copy(buf.at[tidx.at[d]],
                         gbuf.at[pl.ds(base + IDX_CHUNK, IDX_CHUNK)], sem)
        pltpu.async_copy(rbuf.at[ridx.at[d]],
                         gbuf.at[pl.ds(base + 2 * IDX_CHUNK, IDX_CHUNK)], sem)

    def wait_chunk(slot):
        pltpu.make_async_copy(ent_hbm.at[0, pl.ds(0, GW)],
                              gbuf.at[pl.ds(slot * GW, GW)], g_sems[slot]).wait()

    def process_j(jj, buf_id):
        buf = ent_bufs[buf_id]
        rbuf = rel_bufs[buf_id]
        wait_row(jj, buf, rbuf, row_sems[buf_id])
        plsc.subcore_barrier()

        issue_chunk(buf, rbuf, 0, 0)
        for ch in range(NCH):
            slot = ch % 2
            if ch + 1 < NCH:
                issue_chunk(buf, rbuf, ch + 1, (ch + 1) % 2)
            wait_chunk(slot)
            base = slot * GW
            for i in range(IDX_CHUNK // 16):
                hv = gbuf[pl.ds(base + i * 16, 16)]
                tv = gbuf[pl.ds(base + IDX_CHUNK + i * 16, 16)]
                rv = gbuf[pl.ds(base + 2 * IDX_CHUNK + i * 16, 16)]
                dd = hv + rv - tv
                a = pl.ds(ch * IDX_CHUNK + i * 16, 16)
                acc[a] = acc[a] + dd * dd

        plsc.subcore_barrier()

        @pl.when(jj + 2 < J_PER_SC)
        def _():
            issue_row(jj + 2, buf, rbuf, row_sems[buf_id])

    def loop2(i, carry):
        process_j(i * 2, 0)
        process_j(i * 2 + 1, 1)
        return carry

    lax.fori_loop(0, J_PER_SC // 2, loop2, 0)

    @pl.when(c == 0)
    def _():
        pltpu.sync_copy(acc, p0_hbm.at[pl.ds(b0, B_PER_SUB)])

    @pl.when(c == 1)
    def _():
        pltpu.sync_copy(acc, p1_hbm.at[pl.ds(b0, B_PER_SUB)])


def _combine_body(p0_ref, p1_ref, o_ref):
    o_ref[...] = jnp.sqrt(p0_ref[...] + p1_ref[...] + 1e-12)


@jax.jit
def _transe(emb_ent, emb_rel, h, r, t):
    ent_t = emb_ent.T
    rel_t = jnp.pad(emb_rel.T, ((0, 0), (0, RELW - emb_rel.shape[0])))
    mesh = plsc.VectorSubcoreMesh(core_axis_name="c", subcore_axis_name="s",
                                  num_cores=NSC, num_subcores=NSUB)
    f = pl.kernel(
        _sc_body,
        out_type=(jax.ShapeDtypeStruct((BATCH,), jnp.float32),
                  jax.ShapeDtypeStruct((BATCH,), jnp.float32)),
        mesh=mesh,
        compiler_params=pltpu.CompilerParams(
            use_tc_tiling_on_sc=True,
            needs_layout_passes=False,
        ),
        scratch_types=[
            pltpu.VMEM_SHARED((NENT,), jnp.float32),
            pltpu.VMEM_SHARED((NENT,), jnp.float32),
            pltpu.VMEM_SHARED((RELW,), jnp.float32),
            pltpu.VMEM_SHARED((RELW,), jnp.float32),
            pltpu.VMEM((B_PER_SUB,), jnp.int32),
            pltpu.VMEM((B_PER_SUB,), jnp.int32),
            pltpu.VMEM((B_PER_SUB,), jnp.int32),
            pltpu.VMEM((2 * GW,), jnp.float32),
            pltpu.VMEM((B_PER_SUB,), jnp.float32),
            pltpu.SemaphoreType.DMA,
            pltpu.SemaphoreType.DMA,
            pltpu.SemaphoreType.DMA,
            pltpu.SemaphoreType.DMA,
        ],
    )
    p0, p1 = f(ent_t, rel_t, h, r, t)
    out = pl.pallas_call(
        _combine_body,
        out_shape=jax.ShapeDtypeStruct((128, 128), jnp.float32),
    )(p0.reshape(128, 128), p1.reshape(128, 128))
    return out.reshape(BATCH)


def kernel(emb_ent, emb_rel, h, r, t):
    return _transe(emb_ent, emb_rel,
                   h.astype(jnp.int32), r.astype(jnp.int32),
                   t.astype(jnp.int32))
